# Initial kernel scaffold; baseline (speedup 1.0000x reference)
import jax
import jax.numpy as jnp
from jax.experimental import pallas as pl


def kernel(inputs_flatten, embed):
    raise NotImplementedError("write your pallas kernel here")


# speedup vs baseline: 32.0647x; 32.0647x over previous
import functools

import jax
import jax.numpy as jnp
from jax import lax
from jax.experimental import pallas as pl
from jax.experimental.pallas import tpu as pltpu
from jax.experimental.pallas import tpu_sc as plsc

N = 8192
D = 64
K = 8
B = 4096

TILE_B = 256

NC, NS, L = 2, 16, 16
NW = NC * NS
BPW = B // NW


def _topk_tc_body(x_ref, e_ref, idx_ref):
    x = x_ref[...]
    e = e_ref[...]
    mm = lax.dot_general(x, e, (((1,), (1,)), ((), ())),
                         preferred_element_type=jnp.float32)
    e2 = jnp.sum(e * e, axis=1)[None, :]
    xn = jnp.sum(x * x, axis=1, keepdims=True)
    d = (xn + e2) - 2.0 * mm

    col = lax.broadcasted_iota(jnp.int32, (TILE_B, N), 1)
    for k in range(K):
        minv = jnp.min(d, axis=1, keepdims=True)
        idx = jnp.min(jnp.where(d == minv, col, jnp.int32(N)), axis=1)
        idx_ref[k, :] = idx
        d = jnp.where(col == idx[:, None], jnp.inf, d)


def _topk_tc(x, e):
    return pl.pallas_call(
        _topk_tc_body,
        grid=(B // TILE_B,),
        in_specs=[
            pl.BlockSpec((TILE_B, D), lambda i: (i, 0)),
            pl.BlockSpec((N, D), lambda i: (0, 0)),
        ],
        out_specs=pl.BlockSpec((K, TILE_B), lambda i: (0, i)),
        out_shape=jax.ShapeDtypeStruct((K, B), jnp.int32),
    )(x, e)


def _group_sc(idx_t, emb):
    mesh = plsc.VectorSubcoreMesh(core_axis_name="c", subcore_axis_name="s")

    @functools.partial(
        pl.kernel,
        out_type=jax.ShapeDtypeStruct((B, D), jnp.float32),
        mesh=mesh,
        scratch_types=[
            pltpu.VMEM((K, BPW), jnp.int32),
            pltpu.VMEM((K, BPW, D), jnp.float32),
            pltpu.VMEM((BPW, D), jnp.float32),
            pltpu.SemaphoreType.DMA,
        ],
        compiler_params=pltpu.CompilerParams(use_tc_tiling_on_sc=False),
    )
    def body(idx_hbm, emb_hbm, out_hbm, idx_v, rows_v, out_v, sem):
        wid = lax.axis_index("s") * NC + lax.axis_index("c")
        base = wid * BPW
        pltpu.sync_copy(idx_hbm.at[:, pl.ds(base, BPW)], idx_v)
        copies = [pltpu.async_copy(emb_hbm.at[idx_v.at[k]], rows_v.at[k], sem)
                  for k in range(K)]
        for c in copies:
            c.wait()

        def row_body(r, carry):
            for c in range(D // L):
                sl = pl.ds(c * L, L)
                acc = rows_v[0, r, sl]
                acc2 = acc * acc
                for k in range(1, K):
                    v = rows_v[k, r, sl]
                    acc = acc + v
                    acc2 = acc2 + v * v
                out_v[r, sl] = acc2 / acc
            return carry

        lax.fori_loop(0, BPW, row_body, 0)
        pltpu.sync_copy(out_v, out_hbm.at[pl.ds(base, BPW)])

    return body(idx_t, emb)


def kernel(inputs_flatten, embed):
    idx_t = _topk_tc(inputs_flatten, embed)
    group_emb = _group_sc(idx_t, embed)
    return (group_emb, idx_t.T)
